# Initial kernel scaffold; baseline (speedup 1.0000x reference)
import jax
import jax.numpy as jnp
from jax.experimental import pallas as pl


def kernel(voxel_feats_1, voxel_feats_2, edge_index_1, edge_index_2, pc_voxel_id_1, pc_voxel_id_2, w_in, w_enc0, w_enc1, w_enc2, w_enc3, w_enc4, w_enc5, w_dec0, w_dec1, w_dec2, w_dec3, w_dec4, w_dec5, w_g, b_g, w_r, b_r):
    raise NotImplementedError("write your pallas kernel here")


# speedup vs baseline: 1.9109x; 1.9109x over previous
import functools

import jax
import jax.numpy as jnp
from jax import lax
from jax.experimental import pallas as pl
from jax.experimental.pallas import tpu as pltpu
from jax.experimental.pallas import tpu_sc as plsc

N_VOX = 50000
E = 800000
BS = 4
P = 25000

NC, NS, L = 2, 16, 16
NW = NC * NS
EBLK = 128
EPT_G = 196
EPAD = NW * EPT_G * EBLK
CHUNK = 16
R = 512
NPAD = 50176
STRIPE = NPAD // NS
PPB = 25600
PG = (BS * PPB // EBLK) // NW


def _mesh():
    return plsc.VectorSubcoreMesh(core_axis_name="c", subcore_axis_name="s")


_SC_PARAMS = pltpu.CompilerParams(use_tc_tiling_on_sc=False)


@functools.cache
def _deg_kernel():
    scratch = [
        pltpu.VMEM((EBLK,), jnp.int32),
        pltpu.VMEM((EBLK, 16), jnp.float32),
        pltpu.VMEM((STRIPE, 16), jnp.float32),
        pltpu.VMEM_SHARED((NPAD, 16), jnp.float32),
        pltpu.SemaphoreType.DMA,
    ]

    def body(dst_ref, ones_ref, zero_ref, out_ref, didx, ones_v, zbuf, acc, sem):
        core = lax.axis_index("c")
        sub = lax.axis_index("s")
        wid = sub * NC + core
        pltpu.sync_copy(ones_ref, ones_v)
        pltpu.sync_copy(zero_ref, zbuf)
        pltpu.sync_copy(zbuf, acc.at[pl.ds(sub * STRIPE, STRIPE)])
        plsc.subcore_barrier()

        def step(g, carry):
            pltpu.sync_copy(dst_ref.at[wid * EPT_G + g], didx)
            pltpu.sync_copy(ones_v, acc.at[didx], add=True)
            return carry

        lax.fori_loop(0, EPT_G, step, 0)
        plsc.subcore_barrier()
        pltpu.sync_copy(acc.at[pl.ds(sub * STRIPE, STRIPE)],
                        out_ref.at[core, pl.ds(sub * STRIPE, STRIPE)])

    return pl.kernel(
        body,
        out_type=jax.ShapeDtypeStruct((NC, NPAD, 16), jnp.float32),
        mesh=_mesh(),
        scratch_types=scratch,
        compiler_params=_SC_PARAMS,
    )


@functools.cache
def _agg_kernel(nchunk):
    out_t = [jax.ShapeDtypeStruct((NC, NPAD, CHUNK), jnp.float32)] * nchunk
    scratch = [
        pltpu.VMEM((EBLK,), jnp.int32),
        pltpu.VMEM((EBLK,), jnp.int32),
        pltpu.VMEM((EBLK, CHUNK), jnp.float32),
        pltpu.VMEM((STRIPE, CHUNK), jnp.float32),
        pltpu.VMEM_SHARED((NPAD, CHUNK), jnp.float32),
        pltpu.SemaphoreType.DMA,
    ]

    def body(src_ref, dst_ref, zero_ref, *rest):
        h_refs = rest[:nchunk]
        out_refs = rest[nchunk:2 * nchunk]
        sidx, didx, rows, zbuf, acc, sem = rest[2 * nchunk:]
        core = lax.axis_index("c")
        sub = lax.axis_index("s")
        wid = sub * NC + core
        pltpu.sync_copy(zero_ref, zbuf)
        for c in range(nchunk):
            pltpu.sync_copy(zbuf, acc.at[pl.ds(sub * STRIPE, STRIPE)])
            plsc.subcore_barrier()

            def step(g, carry, c=c):
                row = wid * EPT_G + g
                pltpu.sync_copy(src_ref.at[row], sidx)
                pltpu.sync_copy(dst_ref.at[row], didx)
                pltpu.async_copy(h_refs[c].at[sidx], rows, sem).wait()
                pltpu.sync_copy(rows, acc.at[didx], add=True)
                return carry

            lax.fori_loop(0, EPT_G, step, 0)
            plsc.subcore_barrier()
            pltpu.sync_copy(acc.at[pl.ds(sub * STRIPE, STRIPE)],
                            out_refs[c].at[core, pl.ds(sub * STRIPE, STRIPE)])
            if c + 1 < nchunk:
                plsc.subcore_barrier()

    return pl.kernel(body, out_type=out_t, mesh=_mesh(), scratch_types=scratch,
                     compiler_params=_SC_PARAMS)


@functools.cache
def _pool_kernel():
    scratch = [
        pltpu.VMEM((EBLK,), jnp.int32),
        pltpu.VMEM((EBLK, CHUNK), jnp.float32),
        pltpu.VMEM((16,), jnp.float32),
        pltpu.SemaphoreType.DMA,
    ]

    def body(f_ref, pid_ref, out_ref, pidx, rows, accv, sem):
        core = lax.axis_index("c")
        sub = lax.axis_index("s")
        wid = sub * NC + core

        def g_step(g, acc):
            pltpu.sync_copy(pid_ref.at[wid * PG + g], pidx)
            pltpu.async_copy(f_ref.at[pidx], rows, sem).wait()

            def r_step(j, a):
                return a + rows[j, 0:16]

            return lax.fori_loop(0, EBLK, r_step, acc)

        acc = lax.fori_loop(0, PG, g_step, jnp.zeros((16,), jnp.float32))
        accv[...] = acc
        pltpu.sync_copy(accv, out_ref.at[wid])

    return pl.kernel(
        body,
        out_type=jax.ShapeDtypeStruct((NW, 16), jnp.float32),
        mesh=_mesh(),
        scratch_types=scratch,
        compiler_params=_SC_PARAMS,
    )


@functools.cache
def _conv_call(nci, nco, has_skip):
    cpi, cpo = nci * CHUNK, nco * CHUNK
    nsk = nco if has_skip else 0

    def body(*refs):
        hs = refs[:nci]
        aggs = refs[nci:2 * nci]
        deg = refs[2 * nci]
        w = refs[2 * nci + 1]
        skips = refs[2 * nci + 2:2 * nci + 2 + nsk]
        outs = refs[2 * nci + 2 + nsk:]
        h = jnp.concatenate([r[...] for r in hs], axis=1) if nci > 1 else hs[0][...]
        ag = [r[...] for r in aggs]
        agg = jnp.concatenate([a[0] + a[1] for a in ag], axis=1) if nci > 1 \
            else ag[0][0] + ag[0][1]
        d = deg[...]
        dcol = jnp.maximum(d[0, :, 0:1] + d[1, :, 0:1], 1.0)
        z = h + agg / dcol
        y = jnp.maximum(jnp.dot(z, w[...], preferred_element_type=jnp.float32), 0.0)
        for k in range(nco):
            yk = y[:, k * CHUNK:(k + 1) * CHUNK]
            if has_skip:
                yk = yk + skips[k][...]
            outs[k][...] = yk

    in_specs = (
        [pl.BlockSpec((R, CHUNK), lambda i: (i, 0))] * nci
        + [pl.BlockSpec((NC, R, CHUNK), lambda i: (0, i, 0))] * nci
        + [pl.BlockSpec((NC, R, 16), lambda i: (0, i, 0))]
        + [pl.BlockSpec((cpi, cpo), lambda i: (0, 0))]
        + [pl.BlockSpec((R, CHUNK), lambda i: (i, 0))] * nsk
    )
    out_specs = [pl.BlockSpec((R, CHUNK), lambda i: (i, 0))] * nco
    out_shape = [jax.ShapeDtypeStruct((NPAD, CHUNK), jnp.float32)] * nco
    return pl.pallas_call(
        body,
        grid=(NPAD // R,),
        in_specs=in_specs,
        out_specs=out_specs,
        out_shape=out_shape,
    )


def _nch(c):
    return -(-c // CHUNK)


def _pad_w(w, nci, nco):
    wp = jnp.zeros((nci * CHUNK, nco * CHUNK), jnp.float32)
    return wp.at[:w.shape[0], :w.shape[1]].set(w)


def _run_net(x6, edge_index, pcid, weights):
    src = edge_index[0]
    dst = edge_index[1]
    src2d = jnp.concatenate(
        [src, jnp.zeros((EPAD - E,), jnp.int32)]).reshape(EPAD // EBLK, EBLK)
    dst2d = jnp.concatenate(
        [dst, jnp.full((EPAD - E,), NPAD - 1, jnp.int32)]).reshape(EPAD // EBLK, EBLK)
    zero16 = jnp.zeros((STRIPE, 16), jnp.float32)
    zero_c = jnp.zeros((STRIPE, CHUNK), jnp.float32)
    ones16 = jnp.ones((EBLK, 16), jnp.float32)

    deg = _deg_kernel()(dst2d, ones16, zero16)

    h0 = jnp.zeros((NPAD, CHUNK), jnp.float32).at[:N_VOX, :6].set(x6)
    hc = [h0]

    layers = []
    cin = 6
    for w in weights:
        layers.append((cin, w.shape[1], w))
        cin = w.shape[1]

    skips = []
    for li, (ci, co, w) in enumerate(layers):
        nci, nco = _nch(ci), _nch(co)
        wp = _pad_w(w, nci, nco)
        aggs = _agg_kernel(nci)(src2d, dst2d, zero_c, *hc)
        if not isinstance(aggs, (list, tuple)):
            aggs = [aggs]
        is_dec = li >= 7
        skip = skips[12 - li] if is_dec else []
        outs = _conv_call(nci, nco, is_dec)(*hc, *aggs, deg, wp, *skip)
        if not isinstance(outs, (list, tuple)):
            outs = [outs]
        hc = list(outs)
        if li < 6:
            skips.append(hc)

    f = hc[0]
    pid = pcid.reshape(BS, P)
    pid = jnp.concatenate(
        [pid, jnp.full((BS, PPB - P), N_VOX, jnp.int32)], axis=1)
    pid2d = pid.reshape(BS * PPB // EBLK, EBLK)
    pool = _pool_kernel()(f, pid2d)
    return pool.reshape(BS, NW // BS, 16).sum(axis=1) / P


def kernel(voxel_feats_1, voxel_feats_2, edge_index_1, edge_index_2,
           pc_voxel_id_1, pc_voxel_id_2, w_in, w_enc0, w_enc1, w_enc2,
           w_enc3, w_enc4, w_enc5, w_dec0, w_dec1, w_dec2, w_dec3, w_dec4,
           w_dec5, w_g, b_g, w_r, b_r):
    weights = (w_in, w_enc0, w_enc1, w_enc2, w_enc3, w_enc4, w_enc5,
               w_dec0, w_dec1, w_dec2, w_dec3, w_dec4, w_dec5)
    s1 = _run_net(voxel_feats_1, edge_index_1, pc_voxel_id_1, weights)
    s2 = _run_net(voxel_feats_2, edge_index_2, pc_voxel_id_2, weights)
    rg1 = s1 @ w_g + b_g
    rg2 = s2 @ w_g + b_g
    rr1 = s1 @ w_r + b_r
    rr2 = s2 @ w_r + b_r
    return jnp.stack([rg1, rg2, rr1, rr2], axis=0)


# speedup vs baseline: 5.5848x; 2.9227x over previous
import functools

import jax
import jax.numpy as jnp
from jax import lax
from jax.experimental import pallas as pl
from jax.experimental.pallas import tpu as pltpu
from jax.experimental.pallas import tpu_sc as plsc

N_VOX = 50000
E = 800000
BS = 4
P = 25000

NC, NS, L = 2, 16, 16
NW = NC * NS
EBLK = 1792
EPT_G = 14
EPAD = NW * EPT_G * EBLK
CHUNK = 16
R = 512
NPAD = 50176
STRIPE = NPAD // NS
PPB = 25600
PPT = BS * PPB // NW


def _mesh():
    return plsc.VectorSubcoreMesh(core_axis_name="c", subcore_axis_name="s")


_SC_PARAMS = pltpu.CompilerParams(use_tc_tiling_on_sc=False)


@functools.cache
def _deg_kernel():
    scratch = [
        pltpu.VMEM((EBLK,), jnp.int32),
        pltpu.VMEM((EBLK, 16), jnp.float32),
        pltpu.VMEM_SHARED((NPAD, 16), jnp.float32),
        pltpu.SemaphoreType.DMA,
    ]

    def body(dst_ref, ones_ref, zero_ref, out_ref, didx, ones_v, acc, sem):
        core = lax.axis_index("c")
        sub = lax.axis_index("s")
        wid = sub * NC + core
        pltpu.sync_copy(ones_ref, ones_v)
        pltpu.sync_copy(zero_ref, acc.at[pl.ds(sub * STRIPE, STRIPE)])
        plsc.subcore_barrier()

        def step(g, carry):
            pltpu.sync_copy(dst_ref.at[pl.ds((wid * EPT_G + g) * EBLK, EBLK)], didx)
            pltpu.sync_copy(ones_v, acc.at[didx], add=True)
            return carry

        lax.fori_loop(0, EPT_G, step, 0)
        plsc.subcore_barrier()
        pltpu.sync_copy(acc.at[pl.ds(sub * STRIPE, STRIPE)],
                        out_ref.at[core, pl.ds(sub * STRIPE, STRIPE)])

    return pl.kernel(
        body,
        out_type=jax.ShapeDtypeStruct((NC, NPAD, 16), jnp.float32),
        mesh=_mesh(),
        scratch_types=scratch,
        compiler_params=_SC_PARAMS,
    )


@functools.cache
def _agg_kernel(nchunk):
    out_t = [jax.ShapeDtypeStruct((NC, NPAD, CHUNK), jnp.float32)] * nchunk
    scratch = [
        pltpu.VMEM((EBLK,), jnp.int32),
        pltpu.VMEM((EBLK,), jnp.int32),
        pltpu.VMEM((EBLK, CHUNK), jnp.float32),
        pltpu.VMEM_SHARED((NPAD, CHUNK), jnp.float32),
        pltpu.SemaphoreType.DMA,
    ]

    def body(src_ref, dst_ref, zero_ref, *rest):
        h_refs = rest[:nchunk]
        out_refs = rest[nchunk:2 * nchunk]
        sidx, didx, rows, acc, sem = rest[2 * nchunk:]
        core = lax.axis_index("c")
        sub = lax.axis_index("s")
        wid = sub * NC + core
        for c in range(nchunk):
            pltpu.sync_copy(zero_ref, acc.at[pl.ds(sub * STRIPE, STRIPE)])
            plsc.subcore_barrier()

            def step(g, carry, c=c):
                base = (wid * EPT_G + g) * EBLK
                pltpu.sync_copy(src_ref.at[pl.ds(base, EBLK)], sidx)
                pltpu.sync_copy(dst_ref.at[pl.ds(base, EBLK)], didx)
                pltpu.async_copy(h_refs[c].at[sidx], rows, sem).wait()
                pltpu.sync_copy(rows, acc.at[didx], add=True)
                return carry

            lax.fori_loop(0, EPT_G, step, 0)
            plsc.subcore_barrier()
            pltpu.sync_copy(acc.at[pl.ds(sub * STRIPE, STRIPE)],
                            out_refs[c].at[core, pl.ds(sub * STRIPE, STRIPE)])
            if c + 1 < nchunk:
                plsc.subcore_barrier()

    return pl.kernel(body, out_type=out_t, mesh=_mesh(), scratch_types=scratch,
                     compiler_params=_SC_PARAMS)


@functools.cache
def _pool_kernel():
    scratch = [
        pltpu.VMEM((PPT,), jnp.int32),
        pltpu.VMEM((PPT, CHUNK), jnp.float32),
        pltpu.VMEM((16,), jnp.float32),
        pltpu.SemaphoreType.DMA,
    ]

    def body(f_ref, pid_ref, out_ref, pidx, rows, accv, sem):
        core = lax.axis_index("c")
        sub = lax.axis_index("s")
        wid = sub * NC + core
        pltpu.sync_copy(pid_ref.at[pl.ds(wid * PPT, PPT)], pidx)
        pltpu.async_copy(f_ref.at[pidx], rows, sem).wait()

        def r_step(j, a):
            return a + rows[j, 0:16]

        acc = lax.fori_loop(0, PPT, r_step, jnp.zeros((16,), jnp.float32))
        accv[...] = acc
        pltpu.sync_copy(accv, out_ref.at[wid])

    return pl.kernel(
        body,
        out_type=jax.ShapeDtypeStruct((NW, 16), jnp.float32),
        mesh=_mesh(),
        scratch_types=scratch,
        compiler_params=_SC_PARAMS,
    )


@functools.cache
def _conv_call(nci, nco, has_skip):
    cpi, cpo = nci * CHUNK, nco * CHUNK
    nsk = nco if has_skip else 0

    def body(*refs):
        hs = refs[:nci]
        aggs = refs[nci:2 * nci]
        deg = refs[2 * nci]
        w = refs[2 * nci + 1]
        skips = refs[2 * nci + 2:2 * nci + 2 + nsk]
        outs = refs[2 * nci + 2 + nsk:]
        h = jnp.concatenate([r[...] for r in hs], axis=1) if nci > 1 else hs[0][...]
        ag = [r[...] for r in aggs]
        agg = jnp.concatenate([a[0] + a[1] for a in ag], axis=1) if nci > 1 \
            else ag[0][0] + ag[0][1]
        d = deg[...]
        dcol = jnp.maximum(d[0, :, 0:1] + d[1, :, 0:1], 1.0)
        z = h + agg / dcol
        y = jnp.maximum(jnp.dot(z, w[...], preferred_element_type=jnp.float32), 0.0)
        for k in range(nco):
            yk = y[:, k * CHUNK:(k + 1) * CHUNK]
            if has_skip:
                yk = yk + skips[k][...]
            outs[k][...] = yk

    in_specs = (
        [pl.BlockSpec((R, CHUNK), lambda i: (i, 0))] * nci
        + [pl.BlockSpec((NC, R, CHUNK), lambda i: (0, i, 0))] * nci
        + [pl.BlockSpec((NC, R, 16), lambda i: (0, i, 0))]
        + [pl.BlockSpec((cpi, cpo), lambda i: (0, 0))]
        + [pl.BlockSpec((R, CHUNK), lambda i: (i, 0))] * nsk
    )
    out_specs = [pl.BlockSpec((R, CHUNK), lambda i: (i, 0))] * nco
    out_shape = [jax.ShapeDtypeStruct((NPAD, CHUNK), jnp.float32)] * nco
    return pl.pallas_call(
        body,
        grid=(NPAD // R,),
        in_specs=in_specs,
        out_specs=out_specs,
        out_shape=out_shape,
    )


def _nch(c):
    return -(-c // CHUNK)


def _pad_w(w, nci, nco):
    wp = jnp.zeros((nci * CHUNK, nco * CHUNK), jnp.float32)
    return wp.at[:w.shape[0], :w.shape[1]].set(w)


def _run_net(x6, edge_index, pcid, weights):
    src = edge_index[0]
    dst = edge_index[1]
    src1 = jnp.concatenate([src, jnp.zeros((EPAD - E,), jnp.int32)])
    dst1 = jnp.concatenate([dst, jnp.full((EPAD - E,), NPAD - 1, jnp.int32)])
    zero16 = jnp.zeros((STRIPE, 16), jnp.float32)
    ones16 = jnp.ones((EBLK, 16), jnp.float32)

    deg = _deg_kernel()(dst1, ones16, zero16)

    h0 = jnp.zeros((NPAD, CHUNK), jnp.float32).at[:N_VOX, :6].set(x6)
    hc = [h0]

    layers = []
    cin = 6
    for w in weights:
        layers.append((cin, w.shape[1], w))
        cin = w.shape[1]

    skips = []
    for li, (ci, co, w) in enumerate(layers):
        nci, nco = _nch(ci), _nch(co)
        wp = _pad_w(w, nci, nco)
        aggs = _agg_kernel(nci)(src1, dst1, zero16, *hc)
        if not isinstance(aggs, (list, tuple)):
            aggs = [aggs]
        is_dec = li >= 7
        skip = skips[12 - li] if is_dec else []
        outs = _conv_call(nci, nco, is_dec)(*hc, *aggs, deg, wp, *skip)
        if not isinstance(outs, (list, tuple)):
            outs = [outs]
        hc = list(outs)
        if li < 6:
            skips.append(hc)

    f = hc[0]
    pid = pcid.reshape(BS, P)
    pid = jnp.concatenate(
        [pid, jnp.full((BS, PPB - P), N_VOX, jnp.int32)], axis=1).reshape(-1)
    pool = _pool_kernel()(f, pid)
    return pool.reshape(BS, NW // BS, 16).sum(axis=1) / P


def kernel(voxel_feats_1, voxel_feats_2, edge_index_1, edge_index_2,
           pc_voxel_id_1, pc_voxel_id_2, w_in, w_enc0, w_enc1, w_enc2,
           w_enc3, w_enc4, w_enc5, w_dec0, w_dec1, w_dec2, w_dec3, w_dec4,
           w_dec5, w_g, b_g, w_r, b_r):
    weights = (w_in, w_enc0, w_enc1, w_enc2, w_enc3, w_enc4, w_enc5,
               w_dec0, w_dec1, w_dec2, w_dec3, w_dec4, w_dec5)
    s1 = _run_net(voxel_feats_1, edge_index_1, pc_voxel_id_1, weights)
    s2 = _run_net(voxel_feats_2, edge_index_2, pc_voxel_id_2, weights)
    rg1 = s1 @ w_g + b_g
    rg2 = s2 @ w_g + b_g
    rr1 = s1 @ w_r + b_r
    rr2 = s2 @ w_r + b_r
    return jnp.stack([rg1, rg2, rr1, rr2], axis=0)


# speedup vs baseline: 6.1337x; 1.0983x over previous
import functools

import jax
import jax.numpy as jnp
from jax import lax
from jax.experimental import pallas as pl
from jax.experimental.pallas import tpu as pltpu
from jax.experimental.pallas import tpu_sc as plsc

N_VOX = 50000
E = 800000
BS = 4
P = 25000

NC, NS, L = 2, 16, 16
NW = NC * NS
EBLK = 1792
EPT_G = 14
EPAD = NW * EPT_G * EBLK
CHUNK = 16
R = 512
NPAD = 50176
STRIPE = NPAD // NS
PPB = 25600
PPT = BS * PPB // NW


def _mesh():
    return plsc.VectorSubcoreMesh(core_axis_name="c", subcore_axis_name="s")


_SC_PARAMS = pltpu.CompilerParams(use_tc_tiling_on_sc=False)


@functools.cache
def _deg_kernel():
    scratch = [
        pltpu.VMEM((EBLK,), jnp.int32),
        pltpu.VMEM((EBLK, 16), jnp.float32),
        pltpu.VMEM_SHARED((NPAD, 16), jnp.float32),
        pltpu.SemaphoreType.DMA,
    ]

    def body(dst_ref, ones_ref, zero_ref, out_ref, didx, ones_v, acc, sem):
        core = lax.axis_index("c")
        sub = lax.axis_index("s")
        wid = sub * NC + core
        pltpu.sync_copy(ones_ref, ones_v)
        pltpu.sync_copy(zero_ref, acc.at[pl.ds(sub * STRIPE, STRIPE)])
        plsc.subcore_barrier()

        def step(g, carry):
            pltpu.sync_copy(dst_ref.at[pl.ds((wid * EPT_G + g) * EBLK, EBLK)], didx)
            pltpu.sync_copy(ones_v, acc.at[didx], add=True)
            return carry

        lax.fori_loop(0, EPT_G, step, 0)
        plsc.subcore_barrier()
        pltpu.sync_copy(acc.at[pl.ds(sub * STRIPE, STRIPE)],
                        out_ref.at[core, pl.ds(sub * STRIPE, STRIPE)])

    return pl.kernel(
        body,
        out_type=jax.ShapeDtypeStruct((NC, NPAD, 16), jnp.float32),
        mesh=_mesh(),
        scratch_types=scratch,
        compiler_params=_SC_PARAMS,
    )


@functools.cache
def _agg_kernel(nchunk):
    out_t = [jax.ShapeDtypeStruct((NC, NPAD, CHUNK), jnp.float32)] * nchunk
    scratch = [
        pltpu.VMEM((EBLK,), jnp.int32),
        pltpu.VMEM((EBLK,), jnp.int32),
        pltpu.VMEM((EBLK,), jnp.int32),
        pltpu.VMEM((EBLK,), jnp.int32),
        pltpu.VMEM((EBLK, CHUNK), jnp.float32),
        pltpu.VMEM((EBLK, CHUNK), jnp.float32),
        pltpu.VMEM_SHARED((NPAD, CHUNK), jnp.float32),
        pltpu.SemaphoreType.DMA,
        pltpu.SemaphoreType.DMA,
    ]

    def body(src_ref, dst_ref, zero_ref, *rest):
        h_refs = rest[:nchunk]
        out_refs = rest[nchunk:2 * nchunk]
        (sidx0, didx0, sidx1, didx1, rows0, rows1, acc,
         sem0, sem1) = rest[2 * nchunk:]
        core = lax.axis_index("c")
        sub = lax.axis_index("s")
        wid = sub * NC + core
        half = EPT_G // 2
        for c in range(nchunk):
            h = h_refs[c]
            dummy = h.at[pl.ds(0, EBLK)]
            pltpu.sync_copy(zero_ref, acc.at[pl.ds(sub * STRIPE, STRIPE)])
            plsc.subcore_barrier()
            base0 = wid * EPT_G * EBLK
            pltpu.sync_copy(src_ref.at[pl.ds(base0, EBLK)], sidx0)
            pltpu.sync_copy(dst_ref.at[pl.ds(base0, EBLK)], didx0)
            pltpu.async_copy(h.at[sidx0], rows0, sem0)

            def step(i, carry, h=h, dummy=dummy):
                b1 = (wid * EPT_G + 2 * i + 1) * EBLK
                pltpu.sync_copy(src_ref.at[pl.ds(b1, EBLK)], sidx1)
                pltpu.sync_copy(dst_ref.at[pl.ds(b1, EBLK)], didx1)
                pltpu.async_copy(h.at[sidx1], rows1, sem1)
                pltpu.make_async_copy(dummy, rows0, sem0).wait()
                pltpu.sync_copy(rows0, acc.at[didx0], add=True)

                @pl.when(i + 1 < half)
                def _():
                    b2 = (wid * EPT_G + 2 * i + 2) * EBLK
                    pltpu.sync_copy(src_ref.at[pl.ds(b2, EBLK)], sidx0)
                    pltpu.sync_copy(dst_ref.at[pl.ds(b2, EBLK)], didx0)
                    pltpu.async_copy(h.at[sidx0], rows0, sem0)

                pltpu.make_async_copy(dummy, rows1, sem1).wait()
                pltpu.sync_copy(rows1, acc.at[didx1], add=True)
                return carry

            lax.fori_loop(0, half, step, 0)
            plsc.subcore_barrier()
            pltpu.sync_copy(acc.at[pl.ds(sub * STRIPE, STRIPE)],
                            out_refs[c].at[core, pl.ds(sub * STRIPE, STRIPE)])
            if c + 1 < nchunk:
                plsc.subcore_barrier()

    return pl.kernel(body, out_type=out_t, mesh=_mesh(), scratch_types=scratch,
                     compiler_params=_SC_PARAMS)


@functools.cache
def _pool_kernel():
    scratch = [
        pltpu.VMEM((PPT,), jnp.int32),
        pltpu.VMEM((PPT, CHUNK), jnp.float32),
        pltpu.VMEM((16,), jnp.float32),
        pltpu.SemaphoreType.DMA,
    ]

    def body(f_ref, pid_ref, out_ref, pidx, rows, accv, sem):
        core = lax.axis_index("c")
        sub = lax.axis_index("s")
        wid = sub * NC + core
        pltpu.sync_copy(pid_ref.at[pl.ds(wid * PPT, PPT)], pidx)
        pltpu.async_copy(f_ref.at[pidx], rows, sem).wait()

        def r_step(j, a):
            return a + rows[j, 0:16]

        acc = lax.fori_loop(0, PPT, r_step, jnp.zeros((16,), jnp.float32))
        accv[...] = acc
        pltpu.sync_copy(accv, out_ref.at[wid])

    return pl.kernel(
        body,
        out_type=jax.ShapeDtypeStruct((NW, 16), jnp.float32),
        mesh=_mesh(),
        scratch_types=scratch,
        compiler_params=_SC_PARAMS,
    )


@functools.cache
def _conv_call(nci, nco, has_skip):
    cpi, cpo = nci * CHUNK, nco * CHUNK
    nsk = nco if has_skip else 0

    def body(*refs):
        hs = refs[:nci]
        aggs = refs[nci:2 * nci]
        deg = refs[2 * nci]
        w = refs[2 * nci + 1]
        skips = refs[2 * nci + 2:2 * nci + 2 + nsk]
        outs = refs[2 * nci + 2 + nsk:]
        h = jnp.concatenate([r[...] for r in hs], axis=1) if nci > 1 else hs[0][...]
        ag = [r[...] for r in aggs]
        agg = jnp.concatenate([a[0] + a[1] for a in ag], axis=1) if nci > 1 \
            else ag[0][0] + ag[0][1]
        d = deg[...]
        dcol = jnp.maximum(d[0, :, 0:1] + d[1, :, 0:1], 1.0)
        z = h + agg / dcol
        y = jnp.maximum(jnp.dot(z, w[...], preferred_element_type=jnp.float32), 0.0)
        for k in range(nco):
            yk = y[:, k * CHUNK:(k + 1) * CHUNK]
            if has_skip:
                yk = yk + skips[k][...]
            outs[k][...] = yk

    in_specs = (
        [pl.BlockSpec((R, CHUNK), lambda i: (i, 0))] * nci
        + [pl.BlockSpec((NC, R, CHUNK), lambda i: (0, i, 0))] * nci
        + [pl.BlockSpec((NC, R, 16), lambda i: (0, i, 0))]
        + [pl.BlockSpec((cpi, cpo), lambda i: (0, 0))]
        + [pl.BlockSpec((R, CHUNK), lambda i: (i, 0))] * nsk
    )
    out_specs = [pl.BlockSpec((R, CHUNK), lambda i: (i, 0))] * nco
    out_shape = [jax.ShapeDtypeStruct((NPAD, CHUNK), jnp.float32)] * nco
    return pl.pallas_call(
        body,
        grid=(NPAD // R,),
        in_specs=in_specs,
        out_specs=out_specs,
        out_shape=out_shape,
    )


def _nch(c):
    return -(-c // CHUNK)


def _pad_w(w, nci, nco):
    wp = jnp.zeros((nci * CHUNK, nco * CHUNK), jnp.float32)
    return wp.at[:w.shape[0], :w.shape[1]].set(w)


def _run_net(x6, edge_index, pcid, weights):
    src = edge_index[0]
    dst = edge_index[1]
    src1 = jnp.concatenate([src, jnp.zeros((EPAD - E,), jnp.int32)])
    dst1 = jnp.concatenate([dst, jnp.full((EPAD - E,), NPAD - 1, jnp.int32)])
    zero16 = jnp.zeros((STRIPE, 16), jnp.float32)
    ones16 = jnp.ones((EBLK, 16), jnp.float32)

    deg = _deg_kernel()(dst1, ones16, zero16)

    h0 = jnp.zeros((NPAD, CHUNK), jnp.float32).at[:N_VOX, :6].set(x6)
    hc = [h0]

    layers = []
    cin = 6
    for w in weights:
        layers.append((cin, w.shape[1], w))
        cin = w.shape[1]

    skips = []
    for li, (ci, co, w) in enumerate(layers):
        nci, nco = _nch(ci), _nch(co)
        wp = _pad_w(w, nci, nco)
        aggs = _agg_kernel(nci)(src1, dst1, zero16, *hc)
        if not isinstance(aggs, (list, tuple)):
            aggs = [aggs]
        is_dec = li >= 7
        skip = skips[12 - li] if is_dec else []
        outs = _conv_call(nci, nco, is_dec)(*hc, *aggs, deg, wp, *skip)
        if not isinstance(outs, (list, tuple)):
            outs = [outs]
        hc = list(outs)
        if li < 6:
            skips.append(hc)

    f = hc[0]
    pid = pcid.reshape(BS, P)
    pid = jnp.concatenate(
        [pid, jnp.full((BS, PPB - P), N_VOX, jnp.int32)], axis=1).reshape(-1)
    pool = _pool_kernel()(f, pid)
    return pool.reshape(BS, NW // BS, 16).sum(axis=1) / P


def kernel(voxel_feats_1, voxel_feats_2, edge_index_1, edge_index_2,
           pc_voxel_id_1, pc_voxel_id_2, w_in, w_enc0, w_enc1, w_enc2,
           w_enc3, w_enc4, w_enc5, w_dec0, w_dec1, w_dec2, w_dec3, w_dec4,
           w_dec5, w_g, b_g, w_r, b_r):
    weights = (w_in, w_enc0, w_enc1, w_enc2, w_enc3, w_enc4, w_enc5,
               w_dec0, w_dec1, w_dec2, w_dec3, w_dec4, w_dec5)
    s1 = _run_net(voxel_feats_1, edge_index_1, pc_voxel_id_1, weights)
    s2 = _run_net(voxel_feats_2, edge_index_2, pc_voxel_id_2, weights)
    rg1 = s1 @ w_g + b_g
    rg2 = s2 @ w_g + b_g
    rr1 = s1 @ w_r + b_r
    rr2 = s2 @ w_r + b_r
    return jnp.stack([rg1, rg2, rr1, rr2], axis=0)


# speedup vs baseline: 6.1837x; 1.0082x over previous
import functools

import jax
import jax.numpy as jnp
from jax import lax
from jax.experimental import pallas as pl
from jax.experimental.pallas import tpu as pltpu
from jax.experimental.pallas import tpu_sc as plsc

N_VOX = 50000
E = 800000
BS = 4
P = 25000

NC, NS, L = 2, 16, 16
NW = NC * NS
EBLK = 896
EPT_G = 28
EPAD = NW * EPT_G * EBLK
CHUNK = 16
R = 512
NPAD = 50176
STRIPE = NPAD // NS
PPB = 25600
PPT = BS * PPB // NW


def _mesh():
    return plsc.VectorSubcoreMesh(core_axis_name="c", subcore_axis_name="s")


_SC_PARAMS = pltpu.CompilerParams(use_tc_tiling_on_sc=False)


@functools.cache
def _deg_kernel():
    scratch = [
        pltpu.VMEM((EPT_G * EBLK,), jnp.int32),
        pltpu.VMEM((EBLK, 16), jnp.float32),
        pltpu.VMEM_SHARED((NPAD, 16), jnp.float32),
        pltpu.SemaphoreType.DMA,
    ]

    def body(dst_ref, ones_ref, zero_ref, out_ref, didx, ones_v, acc, sem):
        core = lax.axis_index("c")
        sub = lax.axis_index("s")
        wid = sub * NC + core
        pltpu.sync_copy(ones_ref, ones_v)
        pltpu.sync_copy(dst_ref.at[pl.ds(wid * EPT_G * EBLK, EPT_G * EBLK)], didx)
        pltpu.sync_copy(zero_ref, acc.at[pl.ds(sub * STRIPE, STRIPE)])
        plsc.subcore_barrier()

        def step(g, carry):
            pltpu.sync_copy(ones_v, acc.at[didx.at[pl.ds(g * EBLK, EBLK)]], add=True)
            return carry

        lax.fori_loop(0, EPT_G, step, 0)
        plsc.subcore_barrier()
        pltpu.sync_copy(acc.at[pl.ds(sub * STRIPE, STRIPE)],
                        out_ref.at[core, pl.ds(sub * STRIPE, STRIPE)])

    return pl.kernel(
        body,
        out_type=jax.ShapeDtypeStruct((NC, NPAD, 16), jnp.float32),
        mesh=_mesh(),
        scratch_types=scratch,
        compiler_params=_SC_PARAMS,
    )


@functools.cache
def _agg_kernel(nchunk):
    out_t = [jax.ShapeDtypeStruct((NC, NPAD, CHUNK), jnp.float32)] * nchunk
    scratch = [
        pltpu.VMEM((EPT_G * EBLK,), jnp.int32),
        pltpu.VMEM((EPT_G * EBLK,), jnp.int32),
        pltpu.VMEM((EBLK, CHUNK), jnp.float32),
        pltpu.VMEM((EBLK, CHUNK), jnp.float32),
        pltpu.VMEM_SHARED((NPAD, CHUNK), jnp.float32),
        pltpu.SemaphoreType.DMA,
        pltpu.SemaphoreType.DMA,
    ]

    def body(src_ref, dst_ref, zero_ref, *rest):
        h_refs = rest[:nchunk]
        out_refs = rest[nchunk:2 * nchunk]
        sidx, didx, rows0, rows1, acc, sem0, sem1 = rest[2 * nchunk:]
        core = lax.axis_index("c")
        sub = lax.axis_index("s")
        wid = sub * NC + core
        half = EPT_G // 2
        base = wid * EPT_G * EBLK
        pltpu.sync_copy(src_ref.at[pl.ds(base, EPT_G * EBLK)], sidx)
        pltpu.sync_copy(dst_ref.at[pl.ds(base, EPT_G * EBLK)], didx)
        for c in range(nchunk):
            h = h_refs[c]
            dummy = h.at[pl.ds(0, EBLK)]
            pltpu.sync_copy(zero_ref, acc.at[pl.ds(sub * STRIPE, STRIPE)])
            plsc.subcore_barrier()
            pltpu.async_copy(h.at[sidx.at[pl.ds(0, EBLK)]], rows0, sem0)

            def step(i, carry, h=h, dummy=dummy):
                pltpu.async_copy(
                    h.at[sidx.at[pl.ds((2 * i + 1) * EBLK, EBLK)]], rows1, sem1)
                pltpu.make_async_copy(dummy, rows0, sem0).wait()
                pltpu.sync_copy(
                    rows0, acc.at[didx.at[pl.ds(2 * i * EBLK, EBLK)]], add=True)

                @pl.when(i + 1 < half)
                def _(h=h):
                    pltpu.async_copy(
                        h.at[sidx.at[pl.ds((2 * i + 2) * EBLK, EBLK)]], rows0, sem0)

                pltpu.make_async_copy(dummy, rows1, sem1).wait()
                pltpu.sync_copy(
                    rows1, acc.at[didx.at[pl.ds((2 * i + 1) * EBLK, EBLK)]], add=True)
                return carry

            lax.fori_loop(0, half, step, 0)
            plsc.subcore_barrier()
            pltpu.sync_copy(acc.at[pl.ds(sub * STRIPE, STRIPE)],
                            out_refs[c].at[core, pl.ds(sub * STRIPE, STRIPE)])
            if c + 1 < nchunk:
                plsc.subcore_barrier()

    return pl.kernel(body, out_type=out_t, mesh=_mesh(), scratch_types=scratch,
                     compiler_params=_SC_PARAMS)


@functools.cache
def _pool_kernel():
    scratch = [
        pltpu.VMEM((PPT,), jnp.int32),
        pltpu.VMEM((PPT, CHUNK), jnp.float32),
        pltpu.VMEM((16,), jnp.float32),
        pltpu.SemaphoreType.DMA,
    ]

    def body(f_ref, pid_ref, out_ref, pidx, rows, accv, sem):
        core = lax.axis_index("c")
        sub = lax.axis_index("s")
        wid = sub * NC + core
        pltpu.sync_copy(pid_ref.at[pl.ds(wid * PPT, PPT)], pidx)
        pltpu.async_copy(f_ref.at[pidx], rows, sem).wait()

        def r_step(j, a):
            return a + rows[j, 0:16]

        acc = lax.fori_loop(0, PPT, r_step, jnp.zeros((16,), jnp.float32))
        accv[...] = acc
        pltpu.sync_copy(accv, out_ref.at[wid])

    return pl.kernel(
        body,
        out_type=jax.ShapeDtypeStruct((NW, 16), jnp.float32),
        mesh=_mesh(),
        scratch_types=scratch,
        compiler_params=_SC_PARAMS,
    )


@functools.cache
def _conv_call(nci, nco, has_skip):
    cpi, cpo = nci * CHUNK, nco * CHUNK
    nsk = nco if has_skip else 0

    def body(*refs):
        hs = refs[:nci]
        aggs = refs[nci:2 * nci]
        deg = refs[2 * nci]
        w = refs[2 * nci + 1]
        skips = refs[2 * nci + 2:2 * nci + 2 + nsk]
        outs = refs[2 * nci + 2 + nsk:]
        h = jnp.concatenate([r[...] for r in hs], axis=1) if nci > 1 else hs[0][...]
        ag = [r[...] for r in aggs]
        agg = jnp.concatenate([a[0] + a[1] for a in ag], axis=1) if nci > 1 \
            else ag[0][0] + ag[0][1]
        d = deg[...]
        dcol = jnp.maximum(d[0, :, 0:1] + d[1, :, 0:1], 1.0)
        z = h + agg / dcol
        y = jnp.maximum(jnp.dot(z, w[...], preferred_element_type=jnp.float32), 0.0)
        for k in range(nco):
            yk = y[:, k * CHUNK:(k + 1) * CHUNK]
            if has_skip:
                yk = yk + skips[k][...]
            outs[k][...] = yk

    in_specs = (
        [pl.BlockSpec((R, CHUNK), lambda i: (i, 0))] * nci
        + [pl.BlockSpec((NC, R, CHUNK), lambda i: (0, i, 0))] * nci
        + [pl.BlockSpec((NC, R, 16), lambda i: (0, i, 0))]
        + [pl.BlockSpec((cpi, cpo), lambda i: (0, 0))]
        + [pl.BlockSpec((R, CHUNK), lambda i: (i, 0))] * nsk
    )
    out_specs = [pl.BlockSpec((R, CHUNK), lambda i: (i, 0))] * nco
    out_shape = [jax.ShapeDtypeStruct((NPAD, CHUNK), jnp.float32)] * nco
    return pl.pallas_call(
        body,
        grid=(NPAD // R,),
        in_specs=in_specs,
        out_specs=out_specs,
        out_shape=out_shape,
    )


def _nch(c):
    return -(-c // CHUNK)


def _pad_w(w, nci, nco):
    wp = jnp.zeros((nci * CHUNK, nco * CHUNK), jnp.float32)
    return wp.at[:w.shape[0], :w.shape[1]].set(w)


def _run_net(x6, edge_index, pcid, weights):
    src = edge_index[0]
    dst = edge_index[1]
    src1 = jnp.concatenate([src, jnp.zeros((EPAD - E,), jnp.int32)])
    dst1 = jnp.concatenate([dst, jnp.full((EPAD - E,), NPAD - 1, jnp.int32)])
    zero16 = jnp.zeros((STRIPE, 16), jnp.float32)
    ones16 = jnp.ones((EBLK, 16), jnp.float32)

    deg = _deg_kernel()(dst1, ones16, zero16)

    h0 = jnp.zeros((NPAD, CHUNK), jnp.float32).at[:N_VOX, :6].set(x6)
    hc = [h0]

    layers = []
    cin = 6
    for w in weights:
        layers.append((cin, w.shape[1], w))
        cin = w.shape[1]

    skips = []
    for li, (ci, co, w) in enumerate(layers):
        nci, nco = _nch(ci), _nch(co)
        wp = _pad_w(w, nci, nco)
        aggs = _agg_kernel(nci)(src1, dst1, zero16, *hc)
        if not isinstance(aggs, (list, tuple)):
            aggs = [aggs]
        is_dec = li >= 7
        skip = skips[12 - li] if is_dec else []
        outs = _conv_call(nci, nco, is_dec)(*hc, *aggs, deg, wp, *skip)
        if not isinstance(outs, (list, tuple)):
            outs = [outs]
        hc = list(outs)
        if li < 6:
            skips.append(hc)

    f = hc[0]
    pid = pcid.reshape(BS, P)
    pid = jnp.concatenate(
        [pid, jnp.full((BS, PPB - P), N_VOX, jnp.int32)], axis=1).reshape(-1)
    pool = _pool_kernel()(f, pid)
    return pool.reshape(BS, NW // BS, 16).sum(axis=1) / P


def kernel(voxel_feats_1, voxel_feats_2, edge_index_1, edge_index_2,
           pc_voxel_id_1, pc_voxel_id_2, w_in, w_enc0, w_enc1, w_enc2,
           w_enc3, w_enc4, w_enc5, w_dec0, w_dec1, w_dec2, w_dec3, w_dec4,
           w_dec5, w_g, b_g, w_r, b_r):
    weights = (w_in, w_enc0, w_enc1, w_enc2, w_enc3, w_enc4, w_enc5,
               w_dec0, w_dec1, w_dec2, w_dec3, w_dec4, w_dec5)
    s1 = _run_net(voxel_feats_1, edge_index_1, pc_voxel_id_1, weights)
    s2 = _run_net(voxel_feats_2, edge_index_2, pc_voxel_id_2, weights)
    rg1 = s1 @ w_g + b_g
    rg2 = s2 @ w_g + b_g
    rr1 = s1 @ w_r + b_r
    rr2 = s2 @ w_r + b_r
    return jnp.stack([rg1, rg2, rr1, rr2], axis=0)
